# Initial kernel scaffold; baseline (speedup 1.0000x reference)
import jax
import jax.numpy as jnp
from jax.experimental import pallas as pl


def kernel(inputs, pooling_argmax):
    raise NotImplementedError("write your pallas kernel here")


# speedup vs baseline: 24.5956x; 24.5956x over previous
import functools

import jax
import jax.numpy as jnp
from jax import lax
from jax.experimental import pallas as pl
from jax.experimental.pallas import tpu as pltpu
from jax.experimental.pallas import tpu_sc as plsc

_B, _H, _W, _C = 4, 192, 192, 96
_NP = _B * _C
_EPP = _H * _W
_OUT_PLANE = 384 * 384
_HALF = _OUT_PLANE // 2
_WSZ = 9216
_NWIN = _EPP // _WSZ
_NWORK = 32
_PPT = _NP // _NWORK
_L = 16
_VU = 8

_mesh = plsc.VectorSubcoreMesh(core_axis_name="c", subcore_axis_name="s")
_i32 = jnp.int32


@functools.partial(
    pl.kernel,
    mesh=_mesh,
    out_type=jax.ShapeDtypeStruct((_NP, 2, _HALF), jnp.float32),
    scratch_types=[
        pltpu.VMEM((2, _WSZ), jnp.int32),
        pltpu.VMEM((2, _WSZ), jnp.float32),
        pltpu.VMEM((_HALF,), jnp.float32),
    ],
    compiler_params=pltpu.CompilerParams(needs_layout_passes=False),
)
def _unpool_sc(val_hbm, idx_hbm, out_hbm, idx_v, val_v, acc_v):
    wid = (lax.axis_index("s") * 2 + lax.axis_index("c")).astype(jnp.int32)
    zeros = jnp.zeros((_L,), jnp.float32)
    third = jnp.float32(0.33333334)

    def plane_body(j, carry):
        p = wid * _PPT + j

        def half_body(h, carry):
            lo = h * _i32(_HALF)

            def zero_body(i, carry):
                base = i * (_L * _VU)
                for u in range(_VU):
                    acc_v[pl.ds(base + u * _L, _L)] = zeros
                return carry

            lax.fori_loop(_i32(0), _i32(_HALF // (_L * _VU)), zero_body, _i32(0))

            def win_body(w, carry):
                src = pl.ds(w * _WSZ, _WSZ)
                pltpu.sync_copy(idx_hbm.at[p, src], idx_v.at[_i32(0)])
                pltpu.sync_copy(val_hbm.at[p, src], val_v.at[_i32(0)])

                def vec_body(i, carry):
                    base = i * (_L * _VU)
                    for u in range(_VU):
                        s = pl.ds(base + u * _L, _L)
                        a = idx_v[_i32(0), s]
                        t = lax.shift_right_logical(a, _i32(5))
                        q = (t.astype(jnp.float32) * third).astype(jnp.int32)
                        loc = q - lo
                        mask = (loc >= _i32(0)) & (loc < _i32(_HALF))
                        safe = jnp.where(mask, loc, _i32(0))
                        v = val_v[_i32(0), s]
                        plsc.addupdate_scatter(acc_v, [safe], v, mask=mask)
                    return carry

                lax.fori_loop(_i32(0), _i32(_WSZ // (_L * _VU)), vec_body, _i32(0))
                return carry

            lax.fori_loop(_i32(0), _i32(_NWIN), win_body, _i32(0))
            pltpu.sync_copy(acc_v, out_hbm.at[p, h])
            return carry

        lax.fori_loop(_i32(0), _i32(2), half_body, _i32(0))
        return carry

    lax.fori_loop(_i32(0), _i32(_PPT), plane_body, _i32(0))


def kernel(inputs, pooling_argmax):
    idx32 = pooling_argmax.astype(jnp.int32)
    val_t = jnp.transpose(inputs, (0, 3, 1, 2)).reshape(_NP, _EPP)
    idx_t = jnp.transpose(idx32, (0, 3, 1, 2)).reshape(_NP, _EPP)
    out = _unpool_sc(val_t, idx_t)
    out = out.reshape(_B, _C, 384, 384)
    return jnp.transpose(out, (0, 2, 3, 1))


# speedup vs baseline: 41.4498x; 1.6852x over previous
import functools

import jax
import jax.numpy as jnp
from jax import lax
from jax.experimental import pallas as pl
from jax.experimental.pallas import tpu as pltpu
from jax.experimental.pallas import tpu_sc as plsc

_B, _H, _W, _C = 4, 192, 192, 96
_NP = _B * _C
_EPP = _H * _W
_OUT_PLANE = 384 * 384
_HALF = _OUT_PLANE // 2
_WSZ = 12288
_NWIN = _EPP // _WSZ
_NWORK = 32
_PPT = _NP // _NWORK
_NPASS = 2 * _PPT
_L = 16
_VU = 8

_i32 = jnp.int32
_mesh = plsc.VectorSubcoreMesh(core_axis_name="c", subcore_axis_name="s")


@functools.partial(
    pl.kernel,
    mesh=_mesh,
    out_type=jax.ShapeDtypeStruct((_NP, 2, _HALF), jnp.float32),
    scratch_types=[
        pltpu.VMEM((2, _WSZ), jnp.int32),
        pltpu.VMEM((2, _WSZ), jnp.float32),
        pltpu.VMEM((_HALF,), jnp.float32),
        pltpu.SemaphoreType.DMA,
        pltpu.SemaphoreType.DMA,
        pltpu.SemaphoreType.DMA,
        pltpu.SemaphoreType.DMA,
        pltpu.SemaphoreType.DMA,
    ],
    compiler_params=pltpu.CompilerParams(needs_layout_passes=False),
)
def _unpool_sc(val_hbm, idx_hbm, out_hbm, idx_v, val_v, acc_v,
               sem_i0, sem_i1, sem_v0, sem_v1, sem_o):
    wid = (lax.axis_index("s") * 2 + lax.axis_index("c")).astype(jnp.int32)
    base_p = wid * _i32(_PPT)
    zeros = jnp.zeros((_L,), jnp.float32)
    third = jnp.float32(0.33333334)
    sem_i = (sem_i0, sem_i1)
    sem_v = (sem_v0, sem_v1)

    def start_in(p, w, buf):
        src = pl.ds(w * _i32(_WSZ), _WSZ)
        pltpu.async_copy(idx_hbm.at[p, src], idx_v.at[_i32(buf)], sem_i[buf])
        pltpu.async_copy(val_hbm.at[p, src], val_v.at[_i32(buf)], sem_v[buf])

    def wait_in(p, buf):
        src = pl.ds(_i32(0), _WSZ)
        pltpu.make_async_copy(
            idx_hbm.at[p, src], idx_v.at[_i32(buf)], sem_i[buf]).wait()
        pltpu.make_async_copy(
            val_hbm.at[p, src], val_v.at[_i32(buf)], sem_v[buf]).wait()

    def pass_body(k, carry):
        p = base_p + lax.shift_right_logical(k, _i32(1))
        h = k & _i32(1)
        lo = h * _i32(_HALF)

        start_in(p, _i32(0), 0)

        @pl.when(k > _i32(0))
        def _():
            pltpu.make_async_copy(acc_v, out_hbm.at[p, h], sem_o).wait()

        def zero_body(i, carry):
            base = i * (_L * _VU)
            for u in range(_VU):
                acc_v[pl.ds(base + u * _L, _L)] = zeros
            return carry

        lax.fori_loop(_i32(0), _i32(_HALF // (_L * _VU)), zero_body, _i32(0))

        def vec_window(buf):
            def vec_body(i, carry):
                base = i * (_L * _VU)
                for u in range(_VU):
                    s = pl.ds(base + u * _L, _L)
                    a = idx_v[_i32(buf), s]
                    t = lax.shift_right_logical(a, _i32(5))
                    q = (t.astype(jnp.float32) * third).astype(jnp.int32)
                    loc = q - lo
                    mask = plsc.bitcast(loc, jnp.uint32) < jnp.uint32(_HALF)
                    v = val_v[_i32(buf), s]
                    plsc.addupdate_scatter(acc_v, [loc], v, mask=mask)
                return carry

            lax.fori_loop(_i32(0), _i32(_WSZ // (_L * _VU)), vec_body,
                          _i32(0))

        start_in(p, _i32(1), 1)
        wait_in(p, 0)
        vec_window(0)
        start_in(p, _i32(2), 0)
        wait_in(p, 1)
        vec_window(1)
        wait_in(p, 0)
        vec_window(0)

        pltpu.async_copy(acc_v, out_hbm.at[p, h], sem_o)
        return carry

    lax.fori_loop(_i32(0), _i32(_NPASS), pass_body, _i32(0))
    pltpu.make_async_copy(
        acc_v, out_hbm.at[_i32(0), _i32(0)], sem_o).wait()


def kernel(inputs, pooling_argmax):
    idx32 = lax.optimization_barrier(pooling_argmax.astype(jnp.int32))
    val_t = jnp.transpose(inputs, (0, 3, 1, 2)).reshape(_NP, _EPP)
    idx_t = jnp.transpose(idx32, (0, 3, 1, 2)).reshape(_NP, _EPP)
    out = _unpool_sc(val_t, idx_t)
    out = out.reshape(_B, _C, 384, 384)
    return jnp.transpose(out, (0, 2, 3, 1))
